# baseline (speedup 1.0000x reference)
import jax
import jax.numpy as jnp
from jax.experimental import pallas as pl
from jax.experimental.pallas import tpu as pltpu
from jax.experimental.pallas import tpu_sc as plsc

_ROWS_PER_STEP = 2
_SPLIT = 5


def kernel(x, table):
    batch, seq = x.shape
    vocab, dim = table.shape
    win = seq // _SPLIT
    num_steps = batch // _ROWS_PER_STEP

    mesh = plsc.VectorSubcoreMesh(core_axis_name="core",
                                  subcore_axis_name="subcore")

    @pl.kernel(
        out_type=jax.ShapeDtypeStruct((batch, seq, dim), table.dtype),
        mesh=mesh,
        scratch_types=[pltpu.SemaphoreType.DMA],
        compiler_params=pltpu.CompilerParams(use_tc_tiling_on_sc=False),
    )
    def gather_kernel(table_hbm, idx_hbm, out_hbm, sem):
        def body(idx_vmem, out_vmem):
            copies = []
            for r in range(_ROWS_PER_STEP):
                for j in range(_SPLIT):
                    copies.append(pltpu.async_copy(
                        table_hbm.at[idx_vmem.at[r, pl.ds(j * win, win)]],
                        out_vmem.at[r, pl.ds(j * win, win)],
                        sem,
                    ))
            for c in copies:
                c.wait()

        pltpu.emit_pipeline(
            body,
            grid=(num_steps,),
            in_specs=[
                pl.BlockSpec((_ROWS_PER_STEP, seq),
                             index_map=lambda i: (i, 0)),
            ],
            out_specs=[
                pl.BlockSpec((_ROWS_PER_STEP, seq, dim),
                             index_map=lambda i: (i, 0, 0)),
            ],
            core_axis_name=("core", "subcore"),
            dimension_semantics=(pltpu.PARALLEL,),
        )(idx_hbm, out_hbm)

    return gather_kernel(table, x)
